# Initial kernel scaffold; baseline (speedup 1.0000x reference)
import jax
import jax.numpy as jnp
from jax.experimental import pallas as pl


def kernel(x_cat, x_num, tables, W1, b1, W2, b2, W3, b3):
    raise NotImplementedError("write your pallas kernel here")


# speedup vs baseline: 7.8671x; 7.8671x over previous
import functools

import jax
import jax.numpy as jnp
from jax import lax
from jax.experimental import pallas as pl
from jax.experimental.pallas import tpu as pltpu
from jax.experimental.pallas import tpu_sc as plsc

NC = 2
NS = 16
NW = NC * NS
C = 128


def _make_gather(R, D, rpw, nchunk):
  mesh = plsc.VectorSubcoreMesh(core_axis_name="c", subcore_axis_name="s")

  @functools.partial(
      pl.kernel,
      mesh=mesh,
      compiler_params=pltpu.CompilerParams(use_tc_tiling_on_sc=False),
      out_type=jax.ShapeDtypeStruct((R, D), jnp.float32),
      scratch_types=[
          pltpu.VMEM((rpw,), jnp.int32),
          pltpu.VMEM((C, D), jnp.float32),
          pltpu.VMEM((C, D), jnp.float32),
          pltpu.SemaphoreType.DMA,
          pltpu.SemaphoreType.DMA,
      ],
  )
  def gather_kernel(table_hbm, idx_hbm, out_hbm, idx_v, rows0, rows1, s0, s1):
    wid = lax.axis_index("s") * NC + lax.axis_index("c")
    base = wid * rpw
    pltpu.sync_copy(idx_hbm.at[pl.ds(base, rpw)], idx_v)

    def gstart(j, buf, sem):
      pltpu.make_async_copy(
          table_hbm.at[idx_v.at[pl.ds(j * C, C)]], buf, sem).start()

    def gwait(buf, sem):
      pltpu.make_async_copy(table_hbm.at[pl.ds(0, C)], buf, sem).wait()

    def store(j, buf):
      pltpu.sync_copy(buf, out_hbm.at[pl.ds(base + j * C, C)])

    gstart(0, rows0, s0)

    def pair(p, carry):
      j = 2 * p
      gwait(rows0, s0)
      gstart(j + 1, rows1, s1)
      store(j, rows0)

      @pl.when(j + 2 < nchunk)
      def _():
        gstart(j + 2, rows0, s0)

      gwait(rows1, s1)
      store(j + 1, rows1)
      return carry

    lax.fori_loop(0, nchunk // 2, pair, 0)

  return gather_kernel


def _mlp_body(cat_ref, num_ref, w1a_ref, w1b_ref, b1_ref, w2_ref, b2_ref,
              w3_ref, b3_ref, out_ref):
  h = jnp.dot(cat_ref[...], w1a_ref[...], preferred_element_type=jnp.float32)
  h = h + jnp.dot(num_ref[...], w1b_ref[...],
                  preferred_element_type=jnp.float32)
  h = jnp.maximum(h + b1_ref[...], 0.0)
  h = jnp.dot(h, w2_ref[...], preferred_element_type=jnp.float32)
  h = jnp.maximum(h + b2_ref[...], 0.0)
  out_ref[...] = jnp.sum(h * w3_ref[...], axis=1, keepdims=True) + b3_ref[...]


def _mlp(cat_features, x_num, W1a, W1b, b1, W2, b2, w3row, b3, blk):
  B = cat_features.shape[0]
  FD = cat_features.shape[1]
  NUMD = x_num.shape[1]
  H1 = W1a.shape[1]
  H2 = W2.shape[1]
  grid = B // blk
  return pl.pallas_call(
      _mlp_body,
      grid=(grid,),
      in_specs=[
          pl.BlockSpec((blk, FD), lambda i: (i, 0)),
          pl.BlockSpec((blk, NUMD), lambda i: (i, 0)),
          pl.BlockSpec((FD, H1), lambda i: (0, 0)),
          pl.BlockSpec((NUMD, H1), lambda i: (0, 0)),
          pl.BlockSpec((1, H1), lambda i: (0, 0)),
          pl.BlockSpec((H1, H2), lambda i: (0, 0)),
          pl.BlockSpec((1, H2), lambda i: (0, 0)),
          pl.BlockSpec((1, H2), lambda i: (0, 0)),
          pl.BlockSpec((1, 1), lambda i: (0, 0)),
      ],
      out_specs=pl.BlockSpec((blk, 1), lambda i: (i, 0)),
      out_shape=jax.ShapeDtypeStruct((B, 1), jnp.float32),
  )(cat_features, x_num, W1a, W1b, b1, W2, b2, w3row, b3)


def kernel(x_cat, x_num, tables, W1, b1, W2, b2, W3, b3):
  B, F = x_cat.shape
  _, V, D = tables.shape
  FD = F * D
  R = B * F
  rpw = R // NW
  nchunk = rpw // C

  flat_idx = (x_cat.astype(jnp.int32)
              + (jnp.arange(F, dtype=jnp.int32) * V)[None, :]).reshape(R)
  table2d = tables.reshape(F * V, D)

  rows = _make_gather(R, D, rpw, nchunk)(table2d, flat_idx)
  cat_features = rows.reshape(B, FD)

  out = _mlp(cat_features, x_num,
             W1[:FD], W1[FD:], b1.reshape(1, -1),
             W2, b2.reshape(1, -1),
             W3.reshape(1, -1), b3.reshape(1, 1), blk=512)
  return out.reshape(B)


# speedup vs baseline: 8.0354x; 1.0214x over previous
import functools

import jax
import jax.numpy as jnp
from jax import lax
from jax.experimental import pallas as pl
from jax.experimental.pallas import tpu as pltpu
from jax.experimental.pallas import tpu_sc as plsc

NC = 2
NS = 16
NW = NC * NS
C = 512


def _make_gather(R, D, rpw, nchunk):
  mesh = plsc.VectorSubcoreMesh(core_axis_name="c", subcore_axis_name="s")

  @functools.partial(
      pl.kernel,
      mesh=mesh,
      compiler_params=pltpu.CompilerParams(use_tc_tiling_on_sc=False),
      out_type=jax.ShapeDtypeStruct((R, D), jnp.float32),
      scratch_types=[
          pltpu.VMEM((rpw,), jnp.int32),
          pltpu.VMEM((C, D), jnp.float32),
          pltpu.VMEM((C, D), jnp.float32),
          pltpu.SemaphoreType.DMA,
          pltpu.SemaphoreType.DMA,
      ],
  )
  def gather_kernel(table_hbm, idx_hbm, out_hbm, idx_v, rows0, rows1, s0, s1):
    wid = lax.axis_index("s") * NC + lax.axis_index("c")
    base = wid * rpw
    pltpu.sync_copy(idx_hbm.at[pl.ds(base, rpw)], idx_v)

    def gstart(j, buf, sem):
      pltpu.make_async_copy(
          table_hbm.at[idx_v.at[pl.ds(j * C, C)]], buf, sem).start()

    def gwait(buf, sem):
      pltpu.make_async_copy(table_hbm.at[pl.ds(0, C)], buf, sem).wait()

    def store(j, buf):
      pltpu.sync_copy(buf, out_hbm.at[pl.ds(base + j * C, C)])

    gstart(0, rows0, s0)

    def pair(p, carry):
      j = 2 * p
      gwait(rows0, s0)
      gstart(j + 1, rows1, s1)
      store(j, rows0)

      @pl.when(j + 2 < nchunk)
      def _():
        gstart(j + 2, rows0, s0)

      gwait(rows1, s1)
      store(j + 1, rows1)
      return carry

    lax.fori_loop(0, nchunk // 2, pair, 0)

  return gather_kernel


def _mlp_body(cat_ref, num_ref, w1a_ref, w1b_ref, b1_ref, w2_ref, b2_ref,
              w3_ref, b3_ref, out_ref):
  h = jnp.dot(cat_ref[...], w1a_ref[...], preferred_element_type=jnp.float32)
  h = h + jnp.dot(num_ref[...], w1b_ref[...],
                  preferred_element_type=jnp.float32)
  h = jnp.maximum(h + b1_ref[...], 0.0)
  h = jnp.dot(h, w2_ref[...], preferred_element_type=jnp.float32)
  h = jnp.maximum(h + b2_ref[...], 0.0)
  out_ref[...] = jnp.sum(h * w3_ref[...], axis=1, keepdims=True) + b3_ref[...]


def _mlp(cat_features, x_num, W1a, W1b, b1, W2, b2, w3row, b3, blk):
  B = cat_features.shape[0]
  FD = cat_features.shape[1]
  NUMD = x_num.shape[1]
  H1 = W1a.shape[1]
  H2 = W2.shape[1]
  grid = B // blk
  return pl.pallas_call(
      _mlp_body,
      grid=(grid,),
      in_specs=[
          pl.BlockSpec((blk, FD), lambda i: (i, 0)),
          pl.BlockSpec((blk, NUMD), lambda i: (i, 0)),
          pl.BlockSpec((FD, H1), lambda i: (0, 0)),
          pl.BlockSpec((NUMD, H1), lambda i: (0, 0)),
          pl.BlockSpec((1, H1), lambda i: (0, 0)),
          pl.BlockSpec((H1, H2), lambda i: (0, 0)),
          pl.BlockSpec((1, H2), lambda i: (0, 0)),
          pl.BlockSpec((1, H2), lambda i: (0, 0)),
          pl.BlockSpec((1, 1), lambda i: (0, 0)),
      ],
      out_specs=pl.BlockSpec((blk, 1), lambda i: (i, 0)),
      out_shape=jax.ShapeDtypeStruct((B, 1), jnp.float32),
  )(cat_features, x_num, W1a, W1b, b1, W2, b2, w3row, b3)


def kernel(x_cat, x_num, tables, W1, b1, W2, b2, W3, b3):
  B, F = x_cat.shape
  _, V, D = tables.shape
  FD = F * D
  R = B * F
  rpw = R // NW
  nchunk = rpw // C

  flat_idx = (x_cat.astype(jnp.int32)
              + (jnp.arange(F, dtype=jnp.int32) * V)[None, :]).reshape(R)
  table2d = tables.reshape(F * V, D)

  rows = _make_gather(R, D, rpw, nchunk)(table2d, flat_idx)
  cat_features = rows.reshape(B, FD)

  out = _mlp(cat_features, x_num,
             W1[:FD], W1[FD:], b1.reshape(1, -1),
             W2, b2.reshape(1, -1),
             W3.reshape(1, -1), b3.reshape(1, 1), blk=512)
  return out.reshape(B)
